# baseline (speedup 1.0000x reference)
import functools

import jax
import jax.numpy as jnp
from jax import lax
from jax.experimental import pallas as pl
from jax.experimental.pallas import tpu as pltpu
from jax.experimental.pallas import tpu_sc as plsc

_VOCAB = 100000
_D = 64
_B = 1024
_CTX = 20

_NC, _NS = 2, 16
_NW = _NC * _NS
_BPW = _B // _NW
_IPW = _BPW * _CTX
_ICHUNK = 128
_NCHUNK = _IPW // _ICHUNK
_LANES = 16


@functools.lru_cache(maxsize=None)
def _make_pool():
    mesh = plsc.VectorSubcoreMesh(
        core_axis_name="c", subcore_axis_name="s",
        num_cores=_NC, num_subcores=_NS,
    )

    @functools.partial(
        pl.kernel,
        mesh=mesh,
        out_type=jax.ShapeDtypeStruct((_B, _D), jnp.float32),
        scratch_types=[
            pltpu.VMEM((_IPW,), jnp.int32),
            pltpu.VMEM((_IPW, _D), jnp.float32),
            pltpu.VMEM((_BPW, _D), jnp.float32),
            pltpu.SemaphoreType.DMA,
        ],
        compiler_params=pltpu.CompilerParams(use_tc_tiling_on_sc=False),
    )
    def _pool(ctx_hbm, table_hbm, out_hbm, idx_v, rows_v, pooled_v, sem):
        wid = lax.axis_index("s") * _NC + lax.axis_index("c")
        pltpu.sync_copy(ctx_hbm.at[pl.ds(wid * _IPW, _IPW)], idx_v)
        copies = [
            pltpu.async_copy(
                table_hbm.at[idx_v.at[pl.ds(j * _ICHUNK, _ICHUNK)]],
                rows_v.at[pl.ds(j * _ICHUNK, _ICHUNK)],
                sem,
            )
            for j in range(_NCHUNK)
        ]
        for cp in copies:
            cp.wait()

        def body(b, carry):
            for d in range(_D // _LANES):
                sl = pl.ds(d * _LANES, _LANES)
                acc = rows_v[b * _CTX, sl]
                for c in range(1, _CTX):
                    acc = acc + rows_v[b * _CTX + c, sl]
                pooled_v[b, sl] = acc * (1.0 / _CTX)
            return carry

        lax.fori_loop(0, _BPW, body, 0)
        pltpu.sync_copy(pooled_v, out_hbm.at[pl.ds(wid * _BPW, _BPW)])

    return _pool


_VB = 1024
_NVB = (_VOCAB + _VB - 1) // _VB


def _project_kernel(p_ref, w_ref, b_ref, o_ref):
    o_ref[...] = lax.dot_general(
        p_ref[...], w_ref[...],
        dimension_numbers=(((1,), (1,)), ((), ())),
        preferred_element_type=jnp.float32,
    ) + b_ref[...]


def _project(pooled, W, b2):
    return pl.pallas_call(
        _project_kernel,
        grid=(_NVB,),
        in_specs=[
            pl.BlockSpec((_B, _D), lambda v: (0, 0)),
            pl.BlockSpec((_VB, _D), lambda v: (v, 0)),
            pl.BlockSpec((1, _VB), lambda v: (0, v)),
        ],
        out_specs=pl.BlockSpec((_B, _VB), lambda v: (0, v)),
        out_shape=jax.ShapeDtypeStruct((_B, _VOCAB), jnp.float32),
        compiler_params=pltpu.CompilerParams(
            dimension_semantics=("parallel",),
        ),
    )(pooled, W, b2)


def kernel(context, emb_table, W, b):
    ctx = context.astype(jnp.int32).reshape(_B * _CTX)
    pooled = _make_pool()(ctx, emb_table)
    return _project(pooled, W, b.reshape(1, _VOCAB))


# speedup vs baseline: 2.2911x; 2.2911x over previous
import functools

import jax
import jax.numpy as jnp
from jax import lax
from jax.experimental import pallas as pl
from jax.experimental.pallas import tpu as pltpu
from jax.experimental.pallas import tpu_sc as plsc

_VOCAB = 100000
_D = 64
_B = 1024
_CTX = 20

_NC, _NS = 2, 16
_NW = _NC * _NS
_BPW = _B // _NW
_IPW = _BPW * _CTX
_ICHUNK = 128
_NCHUNK = _IPW // _ICHUNK
_LANES = 16


@functools.lru_cache(maxsize=None)
def _make_pool():
    mesh = plsc.VectorSubcoreMesh(
        core_axis_name="c", subcore_axis_name="s",
        num_cores=_NC, num_subcores=_NS,
    )

    @functools.partial(
        pl.kernel,
        mesh=mesh,
        out_type=jax.ShapeDtypeStruct((_B, _D), jnp.float32),
        scratch_types=[
            pltpu.VMEM((_IPW,), jnp.int32),
            pltpu.VMEM((_IPW, _D), jnp.float32),
            pltpu.VMEM((_BPW, _D), jnp.float32),
            pltpu.SemaphoreType.DMA,
        ],
        compiler_params=pltpu.CompilerParams(use_tc_tiling_on_sc=False),
    )
    def _pool(ctx_hbm, table_hbm, out_hbm, idx_v, rows_v, pooled_v, sem):
        wid = lax.axis_index("s") * _NC + lax.axis_index("c")
        pltpu.sync_copy(ctx_hbm.at[pl.ds(wid * _IPW, _IPW)], idx_v)
        copies = [
            pltpu.async_copy(
                table_hbm.at[idx_v.at[pl.ds(j * _ICHUNK, _ICHUNK)]],
                rows_v.at[pl.ds(j * _ICHUNK, _ICHUNK)],
                sem,
            )
            for j in range(_NCHUNK)
        ]
        for cp in copies:
            cp.wait()

        def body(b, carry):
            for d in range(_D // _LANES):
                sl = pl.ds(d * _LANES, _LANES)
                acc = rows_v[b * _CTX, sl]
                for c in range(1, _CTX):
                    acc = acc + rows_v[b * _CTX + c, sl]
                pooled_v[b, sl] = acc * (1.0 / _CTX)
            return carry

        lax.fori_loop(0, _BPW, body, 0)
        pltpu.sync_copy(pooled_v, out_hbm.at[pl.ds(wid * _BPW, _BPW)])

    return _pool


_VB = 2048
_NVB = (_VOCAB + _VB - 1) // _VB


def _project_kernel(w_ref, p_ref, b_ref, o_ref):
    o_ref[...] = lax.dot_general(
        w_ref[...], p_ref[...],
        dimension_numbers=(((0,), (1,)), ((), ())),
        preferred_element_type=jnp.float32,
    ) + b_ref[...]


def _project(WT, pooled, b2):
    return pl.pallas_call(
        _project_kernel,
        grid=(_NVB,),
        in_specs=[
            pl.BlockSpec((_D, _VB), lambda v: (0, v)),
            pl.BlockSpec((_B, _D), lambda v: (0, 0)),
            pl.BlockSpec((_VB, 1), lambda v: (v, 0)),
        ],
        out_specs=pl.BlockSpec((_VB, _B), lambda v: (v, 0)),
        out_shape=jax.ShapeDtypeStruct((_VOCAB, _B), jnp.float32),
        compiler_params=pltpu.CompilerParams(
            dimension_semantics=("parallel",),
        ),
    )(WT, pooled, b2)


def kernel(context, emb_table, W, b):
    ctx = context.astype(jnp.int32).reshape(_B * _CTX)
    pooled = _make_pool()(ctx, emb_table)
    outT = _project(W.T, pooled, b.reshape(_VOCAB, 1))
    return outT.T


# speedup vs baseline: 2.8325x; 1.2363x over previous
import functools

import jax
import jax.numpy as jnp
from jax import lax
from jax.experimental import pallas as pl
from jax.experimental.pallas import tpu as pltpu
from jax.experimental.pallas import tpu_sc as plsc

_VOCAB = 100000
_D = 64
_B = 1024
_CTX = 20

_NC, _NS = 2, 16
_NW = _NC * _NS
_BPW = _B // _NW
_IPW = _BPW * _CTX
_ICHUNK = 128
_NCHUNK = _IPW // _ICHUNK
_LANES = 16


@functools.lru_cache(maxsize=None)
def _make_pool():
    mesh = plsc.VectorSubcoreMesh(
        core_axis_name="c", subcore_axis_name="s",
        num_cores=_NC, num_subcores=_NS,
    )

    @functools.partial(
        pl.kernel,
        mesh=mesh,
        out_type=jax.ShapeDtypeStruct((_B, _D), jnp.float32),
        scratch_types=[
            pltpu.VMEM((_IPW,), jnp.int32),
            pltpu.VMEM((_IPW, _D), jnp.float32),
            pltpu.VMEM((_BPW, _D), jnp.float32),
            pltpu.SemaphoreType.DMA,
        ],
        compiler_params=pltpu.CompilerParams(use_tc_tiling_on_sc=False),
    )
    def _pool(ctx_hbm, table_hbm, out_hbm, idx_v, rows_v, pooled_v, sem):
        wid = lax.axis_index("s") * _NC + lax.axis_index("c")
        pltpu.sync_copy(ctx_hbm.at[pl.ds(wid * _IPW, _IPW)], idx_v)
        copies = [
            pltpu.async_copy(
                table_hbm.at[idx_v.at[pl.ds(j * _ICHUNK, _ICHUNK)]],
                rows_v.at[pl.ds(j * _ICHUNK, _ICHUNK)],
                sem,
            )
            for j in range(_NCHUNK)
        ]
        for cp in copies:
            cp.wait()

        def body(b, carry):
            for d in range(_D // _LANES):
                sl = pl.ds(d * _LANES, _LANES)
                acc = rows_v[b * _CTX, sl]
                for c in range(1, _CTX):
                    acc = acc + rows_v[b * _CTX + c, sl]
                pooled_v[b, sl] = acc * (1.0 / _CTX)
            return carry

        lax.fori_loop(0, _BPW, body, 0)
        pltpu.sync_copy(pooled_v, out_hbm.at[pl.ds(wid * _BPW, _BPW)])

    return _pool


_VB = 2048
_NVB = (_VOCAB + _VB - 1) // _VB


def _project_kernel(w_ref, p_ref, b_ref, o_ref):
    o_ref[...] = lax.dot_general(
        w_ref[...], p_ref[...],
        dimension_numbers=(((0,), (1,)), ((), ())),
        preferred_element_type=jnp.float32,
    ) + b_ref[...].T


def _project(WT, pooled, b2):
    return pl.pallas_call(
        _project_kernel,
        grid=(_NVB,),
        in_specs=[
            pl.BlockSpec((_D, _VB), lambda v: (0, v)),
            pl.BlockSpec((_B, _D), lambda v: (0, 0)),
            pl.BlockSpec((1, _VB), lambda v: (0, v)),
        ],
        out_specs=pl.BlockSpec((_VB, _B), lambda v: (v, 0)),
        out_shape=jax.ShapeDtypeStruct((_VOCAB, _B), jnp.float32),
        compiler_params=pltpu.CompilerParams(
            dimension_semantics=("parallel",),
        ),
    )(WT, pooled, b2)


def kernel(context, emb_table, W, b):
    ctx = context.astype(jnp.int32).reshape(_B * _CTX)
    pooled = _make_pool()(ctx, emb_table)
    outT = _project(W.T, pooled, b.reshape(1, _VOCAB))
    return outT.T
